# baseline (speedup 1.0000x reference)
import jax
import jax.numpy as jnp
from jax import lax
from jax.experimental import pallas as pl
from jax.experimental.pallas import tpu as pltpu
from jax.experimental.pallas import tpu_sc as plsc

_EMBED = 64
_SCALE = 8.0
_NBUF = 4
_LANES = 16
_SPLIT = 104


def kernel(inputTensor, table):
    batch, seq = inputTensor.shape
    num_idx = batch * seq
    idx = inputTensor.reshape(num_idx)

    info = plsc.get_sparse_core_info()
    n_workers = info.num_cores * info.num_subcores
    b_per_worker = batch // n_workers
    idx_per_worker = b_per_worker * seq

    mesh = plsc.VectorSubcoreMesh(
        core_axis_name="core", subcore_axis_name="subcore"
    )

    @jax.jit
    @pl.kernel(
        out_type=jax.ShapeDtypeStruct((batch, seq, _EMBED), table.dtype),
        mesh=mesh,
        scratch_types=[
            pltpu.VMEM((idx_per_worker,), jnp.int32),
            pltpu.VMEM((_NBUF, seq, _EMBED), jnp.float32),
            pltpu.SemaphoreType.DMA((_NBUF,)),
            pltpu.SemaphoreType.DMA((_NBUF,)),
        ],
        compiler_params=pltpu.CompilerParams(use_tc_tiling_on_sc=False),
    )
    def gather_scale(table_hbm, idx_hbm, out_hbm, idx_v, buf, gsem, osem):
        wid = lax.axis_index("subcore") * info.num_cores + lax.axis_index("core")
        b0 = wid * b_per_worker
        pltpu.sync_copy(idx_hbm.at[pl.ds(b0 * seq, idx_per_worker)], idx_v)

        def start_gather(t, k):
            off = t * seq
            pltpu.async_copy(
                table_hbm.at[idx_v.at[pl.ds(off, _SPLIT)]],
                buf.at[k].at[pl.ds(0, _SPLIT)],
                gsem.at[k],
            )
            pltpu.async_copy(
                table_hbm.at[idx_v.at[pl.ds(off + _SPLIT, seq - _SPLIT)]],
                buf.at[k].at[pl.ds(_SPLIT, seq - _SPLIT)],
                gsem.at[k],
            )

        def wait_gather(k):
            pltpu.make_async_copy(
                table_hbm.at[pl.ds(0, seq)], buf.at[k], gsem.at[k]
            ).wait()

        def start_out(t, k):
            pltpu.async_copy(buf.at[k], out_hbm.at[b0 + t], osem.at[k])

        def wait_out(k):
            pltpu.make_async_copy(buf.at[k], out_hbm.at[b0], osem.at[k]).wait()

        def scale(k):
            dst = buf.at[k]

            def row(r, carry):
                for c in range(_EMBED // _LANES):
                    sl = pl.ds(c * _LANES, _LANES)
                    dst[r, sl] = dst[r, sl] * _SCALE
                return carry

            lax.fori_loop(0, seq, row, 0)

        for t in range(2):
            start_gather(t, t)

        def turn(j, carry):
            for k in range(_NBUF):
                t = j * _NBUF + k
                k2 = (k + 2) % _NBUF

                wait_gather(k)

                @pl.when(t + 2 < b_per_worker)
                def _start_ahead(t=t, k2=k2):
                    @pl.when(t >= 2)
                    def _free_slot(k2=k2):
                        wait_out(k2)

                    start_gather(t + 2, k2)

                scale(k)
                start_out(t, k)
            return carry

        lax.fori_loop(0, b_per_worker // _NBUF, turn, 0)

        for k in range(_NBUF):
            wait_out(k)

    out = gather_scale(table, idx)
    return out
